# baseline (speedup 1.0000x reference)
import functools

import jax
import jax.numpy as jnp
from jax import lax
from jax.experimental import pallas as pl
from jax.experimental.pallas import tpu as pltpu
from jax.experimental.pallas import tpu_sc as plsc

P = 8
C = 128
HW = 2048
NROWS = 128
L = 16
KV = C // L
VECS = HW // L
NW = 32
ROWS_PER_W = NROWS // NW
SEG = HW + L
CHUNK = 256


def _sc_pool(xr, labr):
    mesh = plsc.VectorSubcoreMesh(core_axis_name="c", subcore_axis_name="s")

    @functools.partial(
        pl.kernel,
        mesh=mesh,
        out_type=jax.ShapeDtypeStruct((NROWS * P * C,), jnp.float32),
        scratch_types=[
            pltpu.VMEM((HW,), jnp.int32),
            pltpu.VMEM((P * SEG + L,), jnp.int32),
            pltpu.VMEM((CHUNK, C), jnp.float32),
            pltpu.VMEM((CHUNK, C), jnp.float32),
            pltpu.VMEM((P * C,), jnp.float32),
            pltpu.SMEM((P,), jnp.int32),
            pltpu.SemaphoreType.DMA,
            pltpu.SemaphoreType.DMA,
            pltpu.SemaphoreType.DMA,
        ],
        compiler_params=pltpu.CompilerParams(needs_layout_passes=False),
    )
    def k(x_hbm, lab_hbm, out_hbm, lab_v, idx_buf, bufa, bufb, out_v,
          cnt_sm, sema, semb, semo):
        wid = lax.axis_index("s") * 2 + lax.axis_index("c")
        iota = lax.iota(jnp.int32, L)
        trash_idx = P * SEG + iota

        @plsc.parallel_loop(0, (P * SEG + L) // L, unroll=4)
        def _init(i):
            idx_buf[pl.ds(i * L, L)] = jnp.zeros((L,), jnp.int32)

        def row_body(rr, _):
            r = wid * ROWS_PER_W + rr
            pltpu.sync_copy(lab_hbm.at[r], lab_v)

            zero = jnp.zeros((L,), jnp.int32)

            @plsc.parallel_loop(0, VECS, unroll=2, carry=(zero,) * P)
            def _build(i, curs):
                lv = lab_v[pl.ds(i * L, L)]
                pix = iota + i * L
                new = []
                for p in range(P):
                    m = lv == p
                    mi = m.astype(jnp.int32)
                    rank = lax.cumsum(mi, axis=0) - 1
                    dest = jnp.where(m, p * SEG + curs[p] + rank, trash_idx)
                    plsc.store_scatter(idx_buf, [dest], pix)
                    new.append(curs[p] + plsc.all_reduce_population_count(m))
                return tuple(new)

            csplat = _build
            for p in range(P):
                cnt_sm[p] = jnp.max(csplat[p])

            xrow = x_hbm.at[r]

            def start_chunk(p, q, buf, sem):
                idx = idx_buf.at[pl.ds(p * SEG + q * CHUNK, CHUNK)]
                pltpu.make_async_copy(xrow.at[idx], buf, sem).start()

            def wait_chunk(p, q, buf, sem):
                idx = idx_buf.at[pl.ds(p * SEG + q * CHUNK, CHUNK)]
                pltpu.make_async_copy(xrow.at[idx], buf, sem).wait()

            def part_body(p, _):
                cnt = cnt_sm[p]
                nch = (cnt + (CHUNK - 1)) // CHUNK

                @pl.when(nch >= 1)
                def _():
                    start_chunk(p, 0, bufa, sema)

                @pl.when(nch >= 2)
                def _():
                    start_chunk(p, 1, bufb, semb)

                def reduce_chunk(q, acc, buf):
                    cs = jnp.minimum(cnt - q * CHUNK, CHUNK)

                    @plsc.parallel_loop(0, cs, unroll=2, carry=acc)
                    def _red(i, a):
                        new = list(a)
                        for kk in range(KV):
                            v = buf[i, pl.ds(kk * L, L)]
                            new[kk] = new[kk] + v
                            new[KV + kk] = jnp.maximum(new[KV + kk], v)
                        return tuple(new)

                    return _red

                def pair_body(i, acc):
                    q0 = i * 2
                    wait_chunk(p, q0, bufa, sema)

                    @pl.when(q0 + 2 < nch)
                    def _():
                        start_chunk(p, q0 + 2, bufa, sema)

                    acc = reduce_chunk(q0, acc, bufa)
                    wait_chunk(p, q0 + 1, bufb, semb)

                    @pl.when(q0 + 3 < nch)
                    def _():
                        start_chunk(p, q0 + 3, bufb, semb)

                    return reduce_chunk(q0 + 1, acc, bufb)

                init = tuple([jnp.zeros((L,), jnp.float32)] * KV
                             + [jnp.full((L,), -100.0, jnp.float32)] * KV)
                acc = lax.fori_loop(0, nch // 2, pair_body, init)

                def tail(acc):
                    q = nch - 1
                    wait_chunk(p, q, bufa, sema)
                    return reduce_chunk(q, acc, bufa)

                acc = lax.cond(lax.rem(nch, 2) == 1, tail, lambda a: a, acc)

                cntf = jnp.full((L,), cnt).astype(jnp.float32)
                live = cntf > 0.0
                denom = jnp.maximum(cntf, 1.0)
                for kk in range(KV):
                    mean = acc[kk] / denom
                    mx = jnp.where(live, acc[KV + kk], 0.0)
                    out_v[pl.ds(p * C + kk * L, L)] = mean + mx
                return 0

            lax.fori_loop(0, P, part_body, 0)

            dst = out_hbm.at[pl.ds(r * P * C, P * C)]
            pltpu.make_async_copy(out_v, dst, semo).start()
            pltpu.make_async_copy(out_v, dst, semo).wait()
            return 0

        lax.fori_loop(0, ROWS_PER_W, row_body, 0)

    return k(xr, labr)


def kernel(x, part_labels):
    n, c, s, h, w = x.shape
    xr = x.transpose(0, 2, 3, 4, 1).reshape(n * s, h * w, c)
    labr = part_labels.reshape(n * s, h * w).astype(jnp.int32)
    pooled = _sc_pool(xr, labr)
    return pooled.reshape(n, s, P, c).transpose(0, 3, 1, 2)


# speedup vs baseline: 1.5316x; 1.5316x over previous
import functools

import jax
import jax.numpy as jnp
from jax import lax
from jax.experimental import pallas as pl
from jax.experimental.pallas import tpu as pltpu
from jax.experimental.pallas import tpu_sc as plsc

P = 8
C = 128
HW = 2048
NROWS = 128
L = 16
KV = C // L
VECS = HW // L
NW = 32
ROWS_PER_W = NROWS // NW
SEG = HW + L
CHUNK = 128


def _sc_pool(xr, labr):
    mesh = plsc.VectorSubcoreMesh(core_axis_name="c", subcore_axis_name="s")

    @functools.partial(
        pl.kernel,
        mesh=mesh,
        out_type=jax.ShapeDtypeStruct((NROWS * P * C,), jnp.float32),
        scratch_types=[
            pltpu.VMEM((HW,), jnp.int32),
            pltpu.VMEM((P * SEG + L,), jnp.int32),
            pltpu.VMEM((CHUNK, C), jnp.float32),
            pltpu.VMEM((CHUNK, C), jnp.float32),
            pltpu.VMEM((P * C,), jnp.float32),
            pltpu.SMEM((P,), jnp.int32),
            pltpu.SemaphoreType.DMA,
            pltpu.SemaphoreType.DMA,
            pltpu.SemaphoreType.DMA,
        ],
        compiler_params=pltpu.CompilerParams(needs_layout_passes=False),
    )
    def k(x_hbm, lab_hbm, out_hbm, lab_v, idx_buf, bufa, bufb, out_v,
          cnt_sm, sema, semb, semo):
        wid = lax.axis_index("s") * 2 + lax.axis_index("c")
        iota = lax.iota(jnp.int32, L)
        trash_idx = P * SEG + iota

        @plsc.parallel_loop(0, (P * SEG + L) // L, unroll=4)
        def _init(i):
            idx_buf[pl.ds(i * L, L)] = jnp.zeros((L,), jnp.int32)

        def row_body(rr, _):
            r = wid * ROWS_PER_W + rr
            pltpu.sync_copy(lab_hbm.at[r], lab_v)

            zero = jnp.zeros((L,), jnp.int32)

            @plsc.parallel_loop(0, VECS, unroll=2, carry=(zero,) * P)
            def _build(i, curs):
                lv = lab_v[pl.ds(i * L, L)]
                pix = iota + i * L
                new = []
                for p in range(P):
                    m = lv == p
                    mi = m.astype(jnp.int32)
                    rank = lax.cumsum(mi, axis=0) - 1
                    dest = jnp.where(m, p * SEG + curs[p] + rank, trash_idx)
                    plsc.store_scatter(idx_buf, [dest], pix)
                    new.append(curs[p] + plsc.all_reduce_population_count(m))
                return tuple(new)

            csplat = _build
            for p in range(P):
                cnt_sm[p] = jnp.max(csplat[p])

            xrow = x_hbm.at[r]

            def start_chunk(p, q, buf, sem):
                idx = idx_buf.at[pl.ds(p * SEG + q * CHUNK, CHUNK)]
                pltpu.make_async_copy(xrow.at[idx], buf, sem).start()

            def wait_chunk(p, q, buf, sem):
                idx = idx_buf.at[pl.ds(p * SEG + q * CHUNK, CHUNK)]
                pltpu.make_async_copy(xrow.at[idx], buf, sem).wait()

            def part_body(p, _):
                cnt = cnt_sm[p]
                nch = (cnt + (CHUNK - 1)) // CHUNK

                @pl.when(nch >= 1)
                def _():
                    start_chunk(p, 0, bufa, sema)

                @pl.when(nch >= 2)
                def _():
                    start_chunk(p, 1, bufb, semb)

                def reduce_chunk(q, acc, buf):
                    cs = jnp.minimum(cnt - q * CHUNK, CHUNK)

                    @plsc.parallel_loop(0, cs, unroll=2, carry=acc)
                    def _red(i, a):
                        new = list(a)
                        for kk in range(KV):
                            v = buf[i, pl.ds(kk * L, L)]
                            new[kk] = new[kk] + v
                            new[KV + kk] = jnp.maximum(new[KV + kk], v)
                        return tuple(new)

                    return _red

                def pair_body(i, acc):
                    q0 = i * 2
                    wait_chunk(p, q0, bufa, sema)

                    @pl.when(q0 + 2 < nch)
                    def _():
                        start_chunk(p, q0 + 2, bufa, sema)

                    acc = reduce_chunk(q0, acc, bufa)
                    wait_chunk(p, q0 + 1, bufb, semb)

                    @pl.when(q0 + 3 < nch)
                    def _():
                        start_chunk(p, q0 + 3, bufb, semb)

                    return reduce_chunk(q0 + 1, acc, bufb)

                init = tuple([jnp.zeros((L,), jnp.float32)] * KV
                             + [jnp.full((L,), -100.0, jnp.float32)] * KV)
                acc = lax.fori_loop(0, nch // 2, pair_body, init)

                def tail(acc):
                    q = nch - 1
                    wait_chunk(p, q, bufa, sema)
                    return reduce_chunk(q, acc, bufa)

                acc = lax.cond(lax.rem(nch, 2) == 1, tail, lambda a: a, acc)

                cntf = jnp.full((L,), cnt).astype(jnp.float32)
                live = cntf > 0.0
                denom = jnp.maximum(cntf, 1.0)
                for kk in range(KV):
                    mean = acc[kk] / denom
                    mx = jnp.where(live, acc[KV + kk], 0.0)
                    out_v[pl.ds(p * C + kk * L, L)] = mean + mx
                return 0

            lax.fori_loop(0, P, part_body, 0)

            dst = out_hbm.at[pl.ds(r * P * C, P * C)]
            pltpu.make_async_copy(out_v, dst, semo).start()
            pltpu.make_async_copy(out_v, dst, semo).wait()
            return 0

        lax.fori_loop(0, ROWS_PER_W, row_body, 0)

    return k(xr, labr)


def kernel(x, part_labels):
    n, c, s, h, w = x.shape
    xr = x.transpose(0, 2, 3, 4, 1).reshape(n * s, h * w, c)
    labr = part_labels.reshape(n * s, h * w).astype(jnp.int32)
    pooled = _sc_pool(xr, labr)
    return pooled.reshape(n, s, P, c).transpose(0, 3, 1, 2)
